# baseline (speedup 1.0000x reference)
import functools

import jax
import jax.numpy as jnp
from jax import lax
from jax.experimental import pallas as pl
from jax.experimental.pallas import tpu as pltpu
from jax.experimental.pallas import tpu_sc as plsc

G, NM, EM, DF = 10000, 32, 128, 64
NMAC, EMAC = 10000, 160000
GCN1, GCN2, DENSE1, DENSE2 = 64, 64, 32, 4
MACRO_GCN, N_LABELS = 128, 16

B = 4
CH = 25
BG = B * CH
NB = G // BG
BN = B * NM
BE = B * EM
AW = B * DENSE2
DEMB = DENSE2 * GCN2

CHUNK = 128
EPAD = 163840
NACC = 10240
NSC, NTPS = 2, 16
CPT = EPAD // (NSC * NTPS * CHUNK)
RPT = NACC // NTPS


def _split_hl(a):
    hi = a.astype(jnp.bfloat16)
    lo = (a - hi.astype(jnp.float32)).astype(jnp.bfloat16)
    return hi, lo


def _mm3(x, wh, wl):
    return (jnp.dot(x, wh.astype(jnp.float32) + wl.astype(jnp.float32),
                    preferred_element_type=jnp.float32))


def _chain(fr, srow, drow, w1h, w1l, w2h, w2l, fw1h, fw1l, mw1h, mw1l,
           b1, b2, fb1, fw2, fb2, rsel, nodes, eyebn, gmask, eyeaw):
    one = jnp.bfloat16(1.0)
    zero = jnp.bfloat16(0.0)
    soh = jnp.where(nodes == srow, one, zero)
    doh = jnp.where(nodes == drow, one, zero)
    cnt = lax.dot_general(doh, soh, (((1,), (1,)), ((), ())),
                          preferred_element_type=jnp.float32)
    adj = cnt + eyebn
    deg = jnp.sum(cnt, axis=1, keepdims=True) + 1.0
    dinv = lax.rsqrt(deg)

    def conv(x):
        return dinv * jnp.dot(adj, dinv * x,
                              preferred_element_type=jnp.float32)

    xw1 = _mm3(fr, w1h, w1l)
    h1 = jnp.maximum(conv(xw1) + b1, 0.0)
    h2 = conv(_mm3(h1, w2h, w2l)) + b2

    a1 = jnp.tanh(_mm3(h2, fw1h, fw1l) + fb1)
    logits = jnp.dot(a1, fw2, preferred_element_type=jnp.float32) + fb2

    lg3 = logits.reshape(B, NM, DENSE2)
    mx = jnp.max(lg3, axis=1, keepdims=True)
    ex3 = jnp.exp(lg3 - mx)
    att = (ex3 / jnp.sum(ex3, axis=1, keepdims=True)).reshape(BN, DENSE2)

    aw = jnp.dot(att, rsel, preferred_element_type=jnp.float32) * gmask
    emb_all = lax.dot_general(aw, h2, (((0,), (0,)), ((), ())),
                              preferred_element_type=jnp.float32)
    gram = lax.dot_general(aw, aw, (((0,), (0,)), ((), ())),
                           preferred_element_type=jnp.float32)
    pm = gram - eyeaw
    rn = jnp.sqrt(jnp.sum(pm * pm, axis=1, keepdims=True))
    pen = jnp.sum(rn, axis=0, keepdims=True)

    xw = _mm3(emb_all[0:B, :], mw1h[0:GCN2, :], mw1l[0:GCN2, :])
    for i in range(1, DENSE2):
        xw = xw + _mm3(emb_all[i * B:(i + 1) * B, :],
                       mw1h[i * GCN2:(i + 1) * GCN2, :],
                       mw1l[i * GCN2:(i + 1) * GCN2, :])
    return xw, pen


def _micro_body(feat_ref, src_ref, dst_ref, w1h_ref, w1l_ref, w2h_ref,
                w2l_ref, fw1h_ref, fw1l_ref, mw1h_ref, mw1l_ref, b1_ref,
                b2_ref, fb1_ref, fw2_ref, fb2_ref, rsel_ref,
                nodes_ref, eyebn_ref, gmask_ref, eyeaw_ref,
                xw_ref, pen_ref):
    w1 = w1h_ref[...].astype(jnp.float32) + w1l_ref[...].astype(jnp.float32)
    w2 = w2h_ref[...].astype(jnp.float32) + w2l_ref[...].astype(jnp.float32)
    fw1 = (fw1h_ref[...].astype(jnp.float32)
           + fw1l_ref[...].astype(jnp.float32))
    mw1 = (mw1h_ref[...].astype(jnp.float32)
           + mw1l_ref[...].astype(jnp.float32))
    b1, b2, fb1 = b1_ref[...], b2_ref[...], fb1_ref[...]
    fw2, fb2, rsel = fw2_ref[...], fb2_ref[...], rsel_ref[...]
    nodes, eyebn = nodes_ref[...], eyebn_ref[...]
    gmask, eyeaw = gmask_ref[...], eyeaw_ref[...]
    one = jnp.bfloat16(1.0)
    zero = jnp.bfloat16(0.0)
    R = range(CH)

    def mm(x, w):
        return jnp.dot(x, w, preferred_element_type=jnp.float32)

    frs = [feat_ref[c * B:(c + 1) * B].reshape(BN, DF) for c in R]
    sohs = [jnp.where(nodes == src_ref[0, c:c + 1, :], one, zero) for c in R]
    dohs = [jnp.where(nodes == dst_ref[0, c:c + 1, :], one, zero) for c in R]
    cnts = [lax.dot_general(d, s, (((1,), (1,)), ((), ())),
                            preferred_element_type=jnp.float32)
            for d, s in zip(dohs, sohs)]
    adjs = [c + eyebn for c in cnts]
    dinvs = [lax.rsqrt(jnp.sum(c, axis=1, keepdims=True) + 1.0) for c in cnts]
    xw1s = [mm(f, w1) for f in frs]
    t1s = [dinvs[c] * xw1s[c] for c in R]
    m1s = [mm(adjs[c], t1s[c]) for c in R]
    h1s = [jnp.maximum(dinvs[c] * m1s[c] + b1, 0.0) for c in R]
    xw2s = [mm(h, w2) for h in h1s]
    t2s = [dinvs[c] * xw2s[c] for c in R]
    m2s = [mm(adjs[c], t2s[c]) for c in R]
    h2s = [dinvs[c] * m2s[c] + b2 for c in R]
    a1s = [jnp.tanh(mm(h, fw1) + fb1) for h in h2s]
    lgs = [mm(a, fw2) + fb2 for a in a1s]
    lg3s = [lg.reshape(B, NM, DENSE2) for lg in lgs]
    mxs = [jnp.max(lg3, axis=1, keepdims=True) for lg3 in lg3s]
    ex3s = [jnp.exp(lg3s[c] - mxs[c]) for c in R]
    atts = [(ex3s[c] / jnp.sum(ex3s[c], axis=1, keepdims=True)
             ).reshape(BN, DENSE2) for c in R]
    aws = [mm(att, rsel) * gmask for att in atts]
    embs = [lax.dot_general(aws[c], h2s[c], (((0,), (0,)), ((), ())),
                            preferred_element_type=jnp.float32) for c in R]
    grams = [lax.dot_general(a, a, (((0,), (0,)), ((), ())),
                             preferred_element_type=jnp.float32) for a in aws]
    pms = [g - eyeaw for g in grams]
    rns = [jnp.sqrt(jnp.sum(p * p, axis=1, keepdims=True)) for p in pms]
    pens = [jnp.sum(r, axis=0, keepdims=True) for r in rns]
    xws = [mm(embs[c][0:B, :], mw1[0:GCN2, :]) for c in R]
    for i in range(1, DENSE2):
        xws = [xws[c] + mm(embs[c][i * B:(i + 1) * B, :],
                           mw1[i * GCN2:(i + 1) * GCN2, :]) for c in R]
    for c in R:
        xw_ref[0, c * B:(c + 1) * B, :] = xws[c]
    pen = pens[0]
    for c in range(1, CH):
        pen = pen + pens[c]

    @pl.when(pl.program_id(0) == 0)
    def _():
        pen_ref[...] = jnp.zeros((1, 1), jnp.float32)
    pen_ref[...] = pen_ref[...] + pen


def _micro_call(feats, src_rows, dst_rows, w1h, w1l, w2h, w2l, fw1h, fw1l,
                mw1h, mw1l, b1, b2, fb1, fW2, fb2, rsel, nodes, eyebn,
                gmask, eyeaw):
    full = lambda s: pl.BlockSpec(s, lambda i: tuple(0 for _ in s))
    return pl.pallas_call(
        _micro_body,
        grid=(NB,),
        in_specs=[
            pl.BlockSpec((BG, NM, DF), lambda i: (i, 0, 0)),
            pl.BlockSpec((1, CH, BE), lambda i: (i, 0, 0)),
            pl.BlockSpec((1, CH, BE), lambda i: (i, 0, 0)),
            full((DF, GCN1)), full((DF, GCN1)),
            full((GCN1, GCN2)), full((GCN1, GCN2)),
            full((GCN2, DENSE1)), full((GCN2, DENSE1)),
            full((DEMB, MACRO_GCN)), full((DEMB, MACRO_GCN)),
            full((1, GCN1)), full((1, GCN2)), full((1, DENSE1)),
            full((DENSE1, DENSE2)), full((1, DENSE2)), full((DENSE2, AW)),
            full((BN, 1)), full((BN, BN)), full((BN, AW)), full((AW, AW)),
        ],
        out_specs=[
            pl.BlockSpec((1, BG, MACRO_GCN), lambda i: (i, 0, 0)),
            pl.BlockSpec((1, 1), lambda i: (0, 0)),
        ],
        out_shape=[
            jax.ShapeDtypeStruct((NB, BG, MACRO_GCN), jnp.float32),
            jax.ShapeDtypeStruct((1, 1), jnp.float32),
        ],
    )(feats, src_rows, dst_rows, w1h, w1l, w2h, w2l, fw1h, fw1l, mw1h, mw1l,
      b1, b2, fb1, fW2, fb2, rsel, nodes, eyebn, gmask, eyeaw)


def _sc_scatter_add(table, src2, dst2, width, gather):
    mesh = plsc.VectorSubcoreMesh(core_axis_name="c", subcore_axis_name="s")
    cp = pltpu.CompilerParams(use_tc_tiling_on_sc=(width % 128 == 0))
    @functools.partial(
        pl.kernel,
        out_type=jax.ShapeDtypeStruct((NSC * NACC, width), jnp.float32),
        mesh=mesh,
        compiler_params=cp,
        scratch_types=[
            pltpu.VMEM((CPT, CHUNK), jnp.int32),
            pltpu.VMEM((CPT, CHUNK), jnp.int32),
            pltpu.VMEM((CHUNK, width), jnp.float32),
            pltpu.VMEM((CHUNK, width), jnp.float32),
            pltpu.VMEM_SHARED((NACC, width), jnp.float32),
        ],
    )
    def k(tab_hbm, src_hbm, dst_hbm, out_hbm, sidx, didx, rowbuf, zbuf, acc):
        cid = lax.axis_index("c")
        sid = lax.axis_index("s")

        @pl.loop(0, CHUNK)
        def _(r):
            for j in range(width // 16):
                zbuf[r, pl.ds(j * 16, 16)] = jnp.zeros((16,), jnp.float32)
        if not gather:
            @pl.loop(0, CHUNK)
            def _(r):
                for j in range(width // 16):
                    rowbuf[r, pl.ds(j * 16, 16)] = jnp.ones((16,), jnp.float32)

        @pl.loop(0, RPT // CHUNK)
        def _(j):
            pltpu.sync_copy(zbuf, acc.at[pl.ds(sid * RPT + j * CHUNK, CHUNK)])
        plsc.subcore_barrier()

        base = (cid * NTPS + sid) * CPT
        pltpu.sync_copy(dst_hbm.at[pl.ds(base, CPT)], didx)
        if gather:
            pltpu.sync_copy(src_hbm.at[pl.ds(base, CPT)], sidx)

        @pl.loop(0, CPT)
        def _(i):
            if gather:
                pltpu.sync_copy(tab_hbm.at[sidx.at[i]], rowbuf)
            pltpu.sync_copy(rowbuf, acc.at[didx.at[i]], add=True)
        plsc.subcore_barrier()

        pltpu.sync_copy(acc.at[pl.ds(sid * RPT, RPT)],
                        out_hbm.at[pl.ds(cid * NACC + sid * RPT, RPT)])

    return k(table, src2, dst2).reshape(NSC, NACC, width)


def _rows_spec(r, w):
    return pl.BlockSpec((r, w), lambda i: (i, 0))


def _scale_call(xw, d0, d1):
    r = 2000

    def body(xw_ref, d0_ref, d1_ref, xws_ref, dinv_ref):
        deg = d0_ref[:, 0:1] + d1_ref[:, 0:1] + 1.0
        dinv = lax.rsqrt(deg)
        xws_ref[...] = xw_ref[...] * dinv
        dinv_ref[...] = jnp.broadcast_to(dinv, (r, 16))

    return pl.pallas_call(
        body,
        grid=(NMAC // r,),
        in_specs=[_rows_spec(r, MACRO_GCN), _rows_spec(r, 16), _rows_spec(r, 16)],
        out_specs=[_rows_spec(r, MACRO_GCN), _rows_spec(r, 16)],
        out_shape=[jax.ShapeDtypeStruct((NMAC, MACRO_GCN), jnp.float32),
                   jax.ShapeDtypeStruct((NMAC, 16), jnp.float32)],
    )(xw, d0, d1)


def _layer1_call(a0, a1, xw, dinv, mW2, mb1):
    r = 2000

    def body(a0_ref, a1_ref, xw_ref, dinv_ref, mw2_ref, mb1_ref,
             xw2_ref, xw2s_ref):
        d = dinv_ref[:, 0:1]
        nf1 = d * (a0_ref[...] + a1_ref[...]) + d * d * xw_ref[...] + mb1_ref[...]
        nf1 = jnp.maximum(nf1, 0.0)
        xw2 = jnp.dot(nf1, mw2_ref[...], preferred_element_type=jnp.float32)
        xw2_ref[...] = xw2
        xw2s_ref[...] = xw2 * d

    return pl.pallas_call(
        body,
        grid=(NMAC // r,),
        in_specs=[_rows_spec(r, MACRO_GCN), _rows_spec(r, MACRO_GCN),
                  _rows_spec(r, MACRO_GCN), _rows_spec(r, 16),
                  pl.BlockSpec((MACRO_GCN, N_LABELS), lambda i: (0, 0)),
                  pl.BlockSpec((1, MACRO_GCN), lambda i: (0, 0))],
        out_specs=[_rows_spec(r, N_LABELS), _rows_spec(r, N_LABELS)],
        out_shape=[jax.ShapeDtypeStruct((NMAC, N_LABELS), jnp.float32),
                   jax.ShapeDtypeStruct((NMAC, N_LABELS), jnp.float32)],
    )(a0, a1, xw, dinv, mW2, mb1)


def _layer2_call(a0, a1, xw2, dinv, mb2):
    r = 2000

    def body(a0_ref, a1_ref, xw2_ref, dinv_ref, mb2_ref, out_ref):
        d = dinv_ref[:, 0:1]
        nf2 = d * (a0_ref[...] + a1_ref[...]) + d * d * xw2_ref[...] + mb2_ref[...]
        mx = jnp.max(nf2, axis=1, keepdims=True)
        ex = jnp.exp(nf2 - mx)
        lse = mx + jnp.log(jnp.sum(ex, axis=1, keepdims=True))
        out_ref[...] = nf2 - lse

    return pl.pallas_call(
        body,
        grid=(NMAC // r,),
        in_specs=[_rows_spec(r, N_LABELS), _rows_spec(r, N_LABELS),
                  _rows_spec(r, N_LABELS), _rows_spec(r, 16),
                  pl.BlockSpec((1, N_LABELS), lambda i: (0, 0))],
        out_specs=_rows_spec(r, N_LABELS),
        out_shape=jax.ShapeDtypeStruct((NMAC, N_LABELS), jnp.float32),
    )(a0, a1, xw2, dinv, mb2)


def kernel(features, edges, macro_edges, W1, b1, W2, b2, fW1, fb1, fW2, fb2,
           mW1, mb1, mW2, mb2):
    edges = edges.astype(jnp.int32)
    macro_edges = macro_edges.astype(jnp.int32)
    row = lambda v: v.reshape(1, -1).astype(jnp.float32)

    src = macro_edges[0]
    dst = macro_edges[1]
    npad = EPAD - EMAC
    pi = jnp.arange(npad, dtype=jnp.int32)
    src_p = jnp.concatenate([src, pi % NMAC]).reshape(EPAD // CHUNK, CHUNK)
    dst_p = jnp.concatenate([dst, NMAC + pi % (NACC - NMAC)]).reshape(
        EPAD // CHUNK, CHUNK)

    ones_tab = jnp.zeros((1, 16), jnp.float32)
    degp = _sc_scatter_add(ones_tab, src_p, dst_p, 16, gather=False)

    goffs = (jnp.arange(G, dtype=jnp.int32)[:, None] % B) * NM
    src_rows = (edges[:, 0, :] + goffs).astype(
        jnp.bfloat16).reshape(NB, CH, BE)
    dst_rows = (edges[:, 1, :] + goffs).astype(
        jnp.bfloat16).reshape(NB, CH, BE)
    rsel = (jnp.arange(AW, dtype=jnp.int32)[None, :] // B
            == jnp.arange(DENSE2, dtype=jnp.int32)[:, None]).astype(jnp.float32)
    nodes = jnp.arange(BN, dtype=jnp.int32).astype(jnp.bfloat16).reshape(BN, 1)
    eyebn = jnp.eye(BN, dtype=jnp.float32)
    eyeaw = jnp.eye(AW, dtype=jnp.float32)
    gmask = (jnp.arange(BN, dtype=jnp.int32)[:, None] // NM
             == jnp.arange(AW, dtype=jnp.int32)[None, :] % B).astype(jnp.float32)
    w1h, w1l = _split_hl(W1)
    w2h, w2l = _split_hl(W2)
    fw1h, fw1l = _split_hl(fW1)
    mw1h, mw1l = _split_hl(mW1)
    xw3, pen = _micro_call(features, src_rows, dst_rows, w1h, w1l, w2h, w2l,
                           fw1h, fw1l, mw1h, mw1l, row(b1), row(b2),
                           row(fb1), fW2, row(fb2), rsel,
                           nodes, eyebn, gmask, eyeaw)
    xw = xw3.reshape(G, MACRO_GCN)

    xw_s, dinv = _scale_call(xw, degp[0, :NMAC], degp[1, :NMAC])

    acc1 = _sc_scatter_add(xw_s, src_p, dst_p, MACRO_GCN, gather=True)

    xw2, xw2_s = _layer1_call(acc1[0, :NMAC], acc1[1, :NMAC], xw, dinv,
                              mW2, row(mb1))

    acc2 = _sc_scatter_add(xw2_s, src_p, dst_p, N_LABELS, gather=True)

    predictions = _layer2_call(acc2[0, :NMAC], acc2[1, :NMAC], xw2, dinv,
                               row(mb2))
    penalties = pen[0, 0] / jnp.float32(G)
    return predictions, penalties


# speedup vs baseline: 1.0030x; 1.0030x over previous
import functools

import jax
import jax.numpy as jnp
from jax import lax
from jax.experimental import pallas as pl
from jax.experimental.pallas import tpu as pltpu
from jax.experimental.pallas import tpu_sc as plsc

G, NM, EM, DF = 10000, 32, 128, 64
NMAC, EMAC = 10000, 160000
GCN1, GCN2, DENSE1, DENSE2 = 64, 64, 32, 4
MACRO_GCN, N_LABELS = 128, 16

B = 4
CH = 25
BG = B * CH
NB = G // BG
BN = B * NM
BE = B * EM
AW = B * DENSE2
DEMB = DENSE2 * GCN2

CHUNK = 128
EPAD = 163840
NACC = 10240
NSC, NTPS = 2, 16
CPT = EPAD // (NSC * NTPS * CHUNK)
RPT = NACC // NTPS


def _micro_body(feat_ref, src_ref, dst_ref, w1_ref, w2_ref, fw1_ref,
                mw1_ref, b1_ref, b2_ref, fb1_ref, fw2_ref, fb2_ref, rsel_ref,
                nodes_ref, eyebn_ref, gmask_ref, eyeaw_ref,
                xw_ref, pen_ref):
    w1, w2 = w1_ref[...], w2_ref[...]
    fw1, mw1 = fw1_ref[...], mw1_ref[...]
    b1, b2, fb1 = b1_ref[...], b2_ref[...], fb1_ref[...]
    fw2, fb2, rsel = fw2_ref[...], fb2_ref[...], rsel_ref[...]
    nodes, eyebn = nodes_ref[...], eyebn_ref[...]
    gmask, eyeaw = gmask_ref[...], eyeaw_ref[...]
    one = jnp.bfloat16(1.0)
    zero = jnp.bfloat16(0.0)
    R = range(CH)

    def mm(x, w):
        return jnp.dot(x, w, preferred_element_type=jnp.float32)

    frs = [feat_ref[c * B:(c + 1) * B].reshape(BN, DF) for c in R]
    sohs = [jnp.where(nodes == src_ref[0, c:c + 1, :], one, zero) for c in R]
    dohs = [jnp.where(nodes == dst_ref[0, c:c + 1, :], one, zero) for c in R]
    cnts = [lax.dot_general(d, s, (((1,), (1,)), ((), ())),
                            preferred_element_type=jnp.float32)
            for d, s in zip(dohs, sohs)]
    adjs = [c + eyebn for c in cnts]
    dinvs = [lax.rsqrt(jnp.sum(c, axis=1, keepdims=True) + 1.0) for c in cnts]
    xw1s = [mm(f, w1) for f in frs]
    t1s = [dinvs[c] * xw1s[c] for c in R]
    m1s = [mm(adjs[c], t1s[c]) for c in R]
    h1s = [jnp.maximum(dinvs[c] * m1s[c] + b1, 0.0) for c in R]
    xw2s = [mm(h, w2) for h in h1s]
    t2s = [dinvs[c] * xw2s[c] for c in R]
    m2s = [mm(adjs[c], t2s[c]) for c in R]
    h2s = [dinvs[c] * m2s[c] + b2 for c in R]
    a1s = [jnp.tanh(mm(h, fw1) + fb1) for h in h2s]
    lgs = [mm(a, fw2) + fb2 for a in a1s]
    lg3s = [lg.reshape(B, NM, DENSE2) for lg in lgs]
    mxs = [jnp.max(lg3, axis=1, keepdims=True) for lg3 in lg3s]
    ex3s = [jnp.exp(lg3s[c] - mxs[c]) for c in R]
    atts = [(ex3s[c] / jnp.sum(ex3s[c], axis=1, keepdims=True)
             ).reshape(BN, DENSE2) for c in R]
    aws = [mm(att, rsel) * gmask for att in atts]
    embs = [lax.dot_general(aws[c], h2s[c], (((0,), (0,)), ((), ())),
                            preferred_element_type=jnp.float32) for c in R]
    grams = [lax.dot_general(a, a, (((0,), (0,)), ((), ())),
                             preferred_element_type=jnp.float32) for a in aws]
    pms = [g - eyeaw for g in grams]
    rns = [jnp.sqrt(jnp.sum(p * p, axis=1, keepdims=True)) for p in pms]
    pens = [jnp.sum(r, axis=0, keepdims=True) for r in rns]
    xws = [mm(embs[c][0:B, :], mw1[0:GCN2, :]) for c in R]
    for i in range(1, DENSE2):
        xws = [xws[c] + mm(embs[c][i * B:(i + 1) * B, :],
                           mw1[i * GCN2:(i + 1) * GCN2, :]) for c in R]
    for c in R:
        xw_ref[0, c * B:(c + 1) * B, :] = xws[c]
    pen = pens[0]
    for c in range(1, CH):
        pen = pen + pens[c]

    @pl.when(pl.program_id(0) == 0)
    def _():
        pen_ref[...] = jnp.zeros((1, 1), jnp.float32)
    pen_ref[...] = pen_ref[...] + pen


def _micro_call(feats, src_rows, dst_rows, W1, W2, fW1, mW1, b1, b2, fb1,
                fW2, fb2, rsel, nodes, eyebn, gmask, eyeaw):
    full = lambda s: pl.BlockSpec(s, lambda i: tuple(0 for _ in s))
    return pl.pallas_call(
        _micro_body,
        grid=(NB,),
        in_specs=[
            pl.BlockSpec((BG, NM, DF), lambda i: (i, 0, 0)),
            pl.BlockSpec((1, CH, BE), lambda i: (i, 0, 0)),
            pl.BlockSpec((1, CH, BE), lambda i: (i, 0, 0)),
            full((DF, GCN1)), full((GCN1, GCN2)),
            full((GCN2, DENSE1)), full((DEMB, MACRO_GCN)),
            full((1, GCN1)), full((1, GCN2)), full((1, DENSE1)),
            full((DENSE1, DENSE2)), full((1, DENSE2)), full((DENSE2, AW)),
            full((BN, 1)), full((BN, BN)), full((BN, AW)), full((AW, AW)),
        ],
        out_specs=[
            pl.BlockSpec((1, BG, MACRO_GCN), lambda i: (i, 0, 0)),
            pl.BlockSpec((1, 1), lambda i: (0, 0)),
        ],
        out_shape=[
            jax.ShapeDtypeStruct((NB, BG, MACRO_GCN), jnp.float32),
            jax.ShapeDtypeStruct((1, 1), jnp.float32),
        ],
    )(feats, src_rows, dst_rows, W1, W2, fW1, mW1, b1, b2, fb1, fW2, fb2,
      rsel, nodes, eyebn, gmask, eyeaw)


def _sc_scatter_add(table, src2, dst2, width, gather):
    mesh = plsc.VectorSubcoreMesh(core_axis_name="c", subcore_axis_name="s")
    cp = pltpu.CompilerParams(use_tc_tiling_on_sc=(width % 128 == 0))
    @functools.partial(
        pl.kernel,
        out_type=jax.ShapeDtypeStruct((NSC * NACC, width), jnp.float32),
        mesh=mesh,
        compiler_params=cp,
        scratch_types=[
            pltpu.VMEM((CPT, CHUNK), jnp.int32),
            pltpu.VMEM((CPT, CHUNK), jnp.int32),
            pltpu.VMEM((CHUNK, width), jnp.float32),
            pltpu.VMEM((CHUNK, width), jnp.float32),
            pltpu.VMEM_SHARED((NACC, width), jnp.float32),
        ],
    )
    def k(tab_hbm, src_hbm, dst_hbm, out_hbm, sidx, didx, rowbuf, zbuf, acc):
        cid = lax.axis_index("c")
        sid = lax.axis_index("s")

        @pl.loop(0, CHUNK)
        def _(r):
            for j in range(width // 16):
                zbuf[r, pl.ds(j * 16, 16)] = jnp.zeros((16,), jnp.float32)
        if not gather:
            @pl.loop(0, CHUNK)
            def _(r):
                for j in range(width // 16):
                    rowbuf[r, pl.ds(j * 16, 16)] = jnp.ones((16,), jnp.float32)

        @pl.loop(0, RPT // CHUNK)
        def _(j):
            pltpu.sync_copy(zbuf, acc.at[pl.ds(sid * RPT + j * CHUNK, CHUNK)])
        plsc.subcore_barrier()

        base = (cid * NTPS + sid) * CPT
        pltpu.sync_copy(dst_hbm.at[pl.ds(base, CPT)], didx)
        if gather:
            pltpu.sync_copy(src_hbm.at[pl.ds(base, CPT)], sidx)

        @pl.loop(0, CPT)
        def _(i):
            if gather:
                pltpu.sync_copy(tab_hbm.at[sidx.at[i]], rowbuf)
            pltpu.sync_copy(rowbuf, acc.at[didx.at[i]], add=True)
        plsc.subcore_barrier()

        pltpu.sync_copy(acc.at[pl.ds(sid * RPT, RPT)],
                        out_hbm.at[pl.ds(cid * NACC + sid * RPT, RPT)])

    return k(table, src2, dst2).reshape(NSC, NACC, width)


def _rows_spec(r, w):
    return pl.BlockSpec((r, w), lambda i: (i, 0))


def _scale_call(xw, d0, d1):
    r = 2000

    def body(xw_ref, d0_ref, d1_ref, xws_ref, dinv_ref):
        deg = d0_ref[:, 0:1] + d1_ref[:, 0:1] + 1.0
        dinv = lax.rsqrt(deg)
        xws_ref[...] = xw_ref[...] * dinv
        dinv_ref[...] = jnp.broadcast_to(dinv, (r, 16))

    return pl.pallas_call(
        body,
        grid=(NMAC // r,),
        in_specs=[_rows_spec(r, MACRO_GCN), _rows_spec(r, 16), _rows_spec(r, 16)],
        out_specs=[_rows_spec(r, MACRO_GCN), _rows_spec(r, 16)],
        out_shape=[jax.ShapeDtypeStruct((NMAC, MACRO_GCN), jnp.float32),
                   jax.ShapeDtypeStruct((NMAC, 16), jnp.float32)],
    )(xw, d0, d1)


def _layer1_call(a0, a1, xw, dinv, mW2, mb1):
    r = 2000

    def body(a0_ref, a1_ref, xw_ref, dinv_ref, mw2_ref, mb1_ref,
             xw2_ref, xw2s_ref):
        d = dinv_ref[:, 0:1]
        nf1 = d * (a0_ref[...] + a1_ref[...]) + d * d * xw_ref[...] + mb1_ref[...]
        nf1 = jnp.maximum(nf1, 0.0)
        xw2 = jnp.dot(nf1, mw2_ref[...], preferred_element_type=jnp.float32)
        xw2_ref[...] = xw2
        xw2s_ref[...] = xw2 * d

    return pl.pallas_call(
        body,
        grid=(NMAC // r,),
        in_specs=[_rows_spec(r, MACRO_GCN), _rows_spec(r, MACRO_GCN),
                  _rows_spec(r, MACRO_GCN), _rows_spec(r, 16),
                  pl.BlockSpec((MACRO_GCN, N_LABELS), lambda i: (0, 0)),
                  pl.BlockSpec((1, MACRO_GCN), lambda i: (0, 0))],
        out_specs=[_rows_spec(r, N_LABELS), _rows_spec(r, N_LABELS)],
        out_shape=[jax.ShapeDtypeStruct((NMAC, N_LABELS), jnp.float32),
                   jax.ShapeDtypeStruct((NMAC, N_LABELS), jnp.float32)],
    )(a0, a1, xw, dinv, mW2, mb1)


def _layer2_call(a0, a1, xw2, dinv, mb2):
    r = 2000

    def body(a0_ref, a1_ref, xw2_ref, dinv_ref, mb2_ref, out_ref):
        d = dinv_ref[:, 0:1]
        nf2 = d * (a0_ref[...] + a1_ref[...]) + d * d * xw2_ref[...] + mb2_ref[...]
        mx = jnp.max(nf2, axis=1, keepdims=True)
        ex = jnp.exp(nf2 - mx)
        lse = mx + jnp.log(jnp.sum(ex, axis=1, keepdims=True))
        out_ref[...] = nf2 - lse

    return pl.pallas_call(
        body,
        grid=(NMAC // r,),
        in_specs=[_rows_spec(r, N_LABELS), _rows_spec(r, N_LABELS),
                  _rows_spec(r, N_LABELS), _rows_spec(r, 16),
                  pl.BlockSpec((1, N_LABELS), lambda i: (0, 0))],
        out_specs=_rows_spec(r, N_LABELS),
        out_shape=jax.ShapeDtypeStruct((NMAC, N_LABELS), jnp.float32),
    )(a0, a1, xw2, dinv, mb2)


def kernel(features, edges, macro_edges, W1, b1, W2, b2, fW1, fb1, fW2, fb2,
           mW1, mb1, mW2, mb2):
    edges = edges.astype(jnp.int32)
    macro_edges = macro_edges.astype(jnp.int32)
    row = lambda v: v.reshape(1, -1).astype(jnp.float32)

    src = macro_edges[0]
    dst = macro_edges[1]
    npad = EPAD - EMAC
    pi = jnp.arange(npad, dtype=jnp.int32)
    src_p = jnp.concatenate([src, pi % NMAC]).reshape(EPAD // CHUNK, CHUNK)
    dst_p = jnp.concatenate([dst, NMAC + pi % (NACC - NMAC)]).reshape(
        EPAD // CHUNK, CHUNK)

    ones_tab = jnp.zeros((1, 16), jnp.float32)
    degp = _sc_scatter_add(ones_tab, src_p, dst_p, 16, gather=False)

    goffs = (jnp.arange(G, dtype=jnp.int32)[:, None] % B) * NM
    src_rows = (edges[:, 0, :] + goffs).astype(
        jnp.bfloat16).reshape(NB, CH, BE)
    dst_rows = (edges[:, 1, :] + goffs).astype(
        jnp.bfloat16).reshape(NB, CH, BE)
    rsel = (jnp.arange(AW, dtype=jnp.int32)[None, :] // B
            == jnp.arange(DENSE2, dtype=jnp.int32)[:, None]).astype(jnp.float32)
    nodes = jnp.arange(BN, dtype=jnp.int32).astype(jnp.bfloat16).reshape(BN, 1)
    eyebn = jnp.eye(BN, dtype=jnp.float32)
    eyeaw = jnp.eye(AW, dtype=jnp.float32)
    gmask = (jnp.arange(BN, dtype=jnp.int32)[:, None] // NM
             == jnp.arange(AW, dtype=jnp.int32)[None, :] % B).astype(jnp.float32)
    xw3, pen = _micro_call(features, src_rows, dst_rows, W1, W2, fW1, mW1,
                           row(b1), row(b2), row(fb1), fW2, row(fb2), rsel,
                           nodes, eyebn, gmask, eyeaw)
    xw = xw3.reshape(G, MACRO_GCN)

    xw_s, dinv = _scale_call(xw, degp[0, :NMAC], degp[1, :NMAC])

    acc1 = _sc_scatter_add(xw_s, src_p, dst_p, MACRO_GCN, gather=True)

    xw2, xw2_s = _layer1_call(acc1[0, :NMAC], acc1[1, :NMAC], xw, dinv,
                              mW2, row(mb1))

    acc2 = _sc_scatter_add(xw2_s, src_p, dst_p, N_LABELS, gather=True)

    predictions = _layer2_call(acc2[0, :NMAC], acc2[1, :NMAC], xw2, dinv,
                               row(mb2))
    penalties = pen[0, 0] / jnp.float32(G)
    return predictions, penalties
